# baseline (speedup 1.0000x reference)
import functools

import jax
import jax.numpy as jnp
from jax import lax
from jax.experimental import pallas as pl
from jax.experimental.pallas import tpu as pltpu
from jax.experimental.pallas import tpu_sc as plsc

IN_CH = 256
OUT_CH = 256
N = 10000
E = 160000
NEG = 0.2

HALF = 128
NTILES = 16
CH = 128
QR = 32
NCH = 80
EPT = NCH * CH
EPT_REAL = E // NTILES
N_PAD = 10112
ROWS_PT = N_PAD // NTILES
BM = 1000
NEG_BIG = -1e30


def _tc_body(x_ref, w_ref, a_ref, h2_ref, s_ref):
    xb = x_ref[...]
    w = w_ref[...]
    hb = lax.dot_general(xb, w, (((1,), (0,)), ((), ())),
                         preferred_element_type=jnp.float32,
                         precision=lax.Precision.HIGHEST)
    h2_ref[0, ...] = hb[:, :HALF]
    h2_ref[1, ...] = hb[:, HALF:]
    wa = lax.dot_general(w, a_ref[...], (((1,), (0,)), ((), ())),
                         preferred_element_type=jnp.float32,
                         precision=lax.Precision.HIGHEST)
    s_ref[...] = lax.dot_general(xb, wa, (((1,), (0,)), ((), ())),
                                 preferred_element_type=jnp.float32,
                                 precision=lax.Precision.HIGHEST)


_tc_mm = pl.pallas_call(
    _tc_body,
    grid=(N // BM,),
    in_specs=[
        pl.BlockSpec((BM, IN_CH), lambda i: (i, 0)),
        pl.BlockSpec((IN_CH, OUT_CH), lambda i: (0, 0)),
        pl.BlockSpec((OUT_CH, HALF), lambda i: (0, 0)),
    ],
    out_specs=[
        pl.BlockSpec((2, BM, HALF), lambda i: (0, i, 0)),
        pl.BlockSpec((BM, HALF), lambda i: (i, 0)),
    ],
    out_shape=[
        jax.ShapeDtypeStruct((2, N, HALF), jnp.float32),
        jax.ShapeDtypeStruct((N, HALF), jnp.float32),
    ],
)


def _i16(v):
    return jnp.full((16,), v, jnp.int32)


@functools.partial(
    pl.kernel,
    mesh=plsc.VectorSubcoreMesh(core_axis_name="c", subcore_axis_name="s"),
    compiler_params=pltpu.CompilerParams(needs_layout_passes=False),
    out_type=jax.ShapeDtypeStruct((N, 2 * HALF), jnp.float32),
    scratch_types=[
        pltpu.VMEM((EPT,), jnp.int32),
        pltpu.VMEM((EPT,), jnp.int32),
        pltpu.VMEM((EPT,), jnp.float32),
        pltpu.VMEM((CH, HALF), jnp.float32),
        pltpu.VMEM((QR,), jnp.int32),
        pltpu.VMEM((QR,), jnp.int32),
        pltpu.VMEM((QR,), jnp.int32),
        pltpu.VMEM((QR,), jnp.int32),
        pltpu.VMEM((QR,), jnp.int32),
        pltpu.VMEM((QR,), jnp.int32),
        pltpu.VMEM((QR,), jnp.int32),
        pltpu.VMEM((QR,), jnp.int32),
        pltpu.VMEM((16,), jnp.float32),
        pltpu.VMEM((NTILES * 16,), jnp.float32),
        pltpu.VMEM_SHARED((N_PAD, HALF), jnp.float32),
        pltpu.VMEM_SHARED((NTILES * 16,), jnp.float32),
        pltpu.VMEM_SHARED((NTILES * 16,), jnp.float32),
        pltpu.SemaphoreType.DMA,
        pltpu.SemaphoreType.DMA,
        pltpu.SemaphoreType.DMA,
        pltpu.SemaphoreType.DMA,
        pltpu.SemaphoreType.DMA,
        pltpu.SemaphoreType.DMA,
        pltpu.SemaphoreType.DMA,
        pltpu.SemaphoreType.DMA,
        pltpu.SemaphoreType.DMA,
    ],
)
def _sc_gat(h_hbm, sd_hbm, ss_hbm, dst_hbm, src_hbm, out_hbm,
            dst_v, src_v, coef_v, buf,
            dq0, dq1, dq2, dq3, sq0, sq1, sq2, sq3, pub, redv,
            acc, redm, reds,
            asem, gs0, gs1, gs2, gs3, ss0, ss1, ss2, ss3):
    c = lax.axis_index("c")
    t = lax.axis_index("s")

    pltpu.sync_copy(dst_hbm.at[pl.ds(t * EPT_REAL, EPT_REAL)],
                    dst_v.at[pl.ds(0, EPT_REAL)])
    pltpu.sync_copy(src_hbm.at[pl.ds(t * EPT_REAL, EPT_REAL)],
                    src_v.at[pl.ds(0, EPT_REAL)])
    for w in range((EPT - EPT_REAL) // 16):
        dst_v[pl.ds(EPT_REAL + w * 16, 16)] = _i16(N_PAD - 1)
        src_v[pl.ds(EPT_REAL + w * 16, 16)] = _i16(0)

    zf = jnp.full((16,), 0.0, jnp.float32)

    def _zero_buf(k, _):
        for q in range(HALF // 16):
            buf[k, pl.ds(q * 16, 16)] = zf
        return 0

    lax.fori_loop(0, CH, _zero_buf, 0)

    base = t * ROWS_PT
    nfull = ROWS_PT // CH
    rem = ROWS_PT % CH

    def _zero_acc(k, _):
        pltpu.sync_copy(buf, acc.at[pl.ds(base + k * CH, CH)])
        return 0

    lax.fori_loop(0, nfull, _zero_acc, 0)
    if rem:
        pltpu.sync_copy(buf.at[pl.ds(0, rem)],
                        acc.at[pl.ds(base + nfull * CH, rem)])

    lanes = lax.iota(jnp.int32, 16)
    gbase = t * EPT

    def _fire_scores(j, _):
        pltpu.async_copy(sd_hbm.at[dst_v.at[pl.ds(j * CH, CH)]],
                         coef_v.at[pl.ds(j * CH, CH)], asem)
        pltpu.async_copy(ss_hbm.at[src_v.at[pl.ds(j * CH, CH)]],
                         buf.at[j], asem)
        return 0

    lax.fori_loop(0, NCH, _fire_scores, 0)

    def _drain_scores(j, _):
        pltpu.make_async_copy(sd_hbm.at[pl.ds(0, CH)],
                              coef_v.at[pl.ds(j * CH, CH)], asem).wait()
        pltpu.make_async_copy(ss_hbm.at[pl.ds(0, CH)], buf.at[j],
                              asem).wait()
        return 0

    lax.fori_loop(0, NCH, _drain_scores, 0)

    offv = _i16(c * N)

    def _score_chunk(j, mx):
        for i in range(CH // 16):
            o = j * CH + i * 16
            a = coef_v[pl.ds(o, 16)] + buf[j, pl.ds(i * 16, 16)]
            a = jnp.where(a >= 0.0, a, NEG * a)
            gid = _i16(o) + lanes
            a = jnp.where(gid < EPT_REAL, a, NEG_BIG)
            coef_v[pl.ds(o, 16)] = a
            mx = jnp.maximum(mx, a)
            src_v[pl.ds(o, 16)] = src_v[pl.ds(o, 16)] + offv
        return mx

    mx = lax.fori_loop(0, NCH, _score_chunk,
                       jnp.full((16,), NEG_BIG, jnp.float32))

    def _prep_q(p, colbase, dq, sq):
        for v in range(QR // 16):
            o = p * CH + colbase + v * 16
            dq[pl.ds(v * 16, 16)] = dst_v[pl.ds(o, 16)]
            sq[pl.ds(v * 16, 16)] = src_v[pl.ds(o, 16)]

    def _slotq(s):
        return buf.at[pl.ds(s * QR, QR)]

    _prep_q(0, 0, dq0, sq0)
    pltpu.async_copy(h_hbm.at[sq0], _slotq(0), gs0)
    _prep_q(0, QR, dq1, sq1)
    pltpu.async_copy(h_hbm.at[sq1], _slotq(1), gs1)

    pub[...] = mx
    pltpu.sync_copy(pub, redm.at[pl.ds(t * 16, 16)])
    plsc.subcore_barrier()
    pltpu.sync_copy(redm, redv)
    m16 = redv[pl.ds(0, 16)]
    for i in range(1, NTILES):
        m16 = jnp.maximum(m16, redv[pl.ds(i * 16, 16)])
    mvec = jnp.full((16,), jnp.max(m16))

    def _exp_chunk(j, sm):
        for i in range(CH // 16):
            e = jnp.exp(coef_v[pl.ds(j * CH + i * 16, 16)] - mvec)
            coef_v[pl.ds(j * CH + i * 16, 16)] = e
            sm = sm + e
        return sm

    sm = lax.fori_loop(0, NCH, _exp_chunk, jnp.full((16,), 0.0, jnp.float32))

    pub[...] = sm
    pltpu.sync_copy(pub, reds.at[pl.ds(t * 16, 16)])
    plsc.subcore_barrier()
    pltpu.sync_copy(reds, redv)
    s16 = redv[pl.ds(0, 16)]
    for i in range(1, NTILES):
        s16 = s16 + redv[pl.ds(i * 16, 16)]
    ivec = jnp.full((16,), 1.0, jnp.float32) / jnp.full((16,), jnp.sum(s16))

    dqs = (dq0, dq1, dq2, dq3)
    sqs = (sq0, sq1, sq2, sq3)
    gss = (gs0, gs1, gs2, gs3)
    sss = (ss0, ss1, ss2, ss3)

    def _wait_g(s):
        pltpu.make_async_copy(h_hbm.at[pl.ds(0, QR)], _slotq(s),
                              gss[s]).wait()

    def _wait_s(s):
        pltpu.make_async_copy(h_hbm.at[pl.ds(0, QR)], _slotq(s),
                              sss[s]).wait()

    def _scaleq(p, colbase, s):
        def _row4(k4, _):
            k = k4 * 4
            for d in range(4):
                bc = plsc.load_gather(
                    coef_v, [_i16(p * CH + colbase + k + d)]) * ivec
                r = s * QR + k + d
                for q in range(HALF // 16):
                    buf[r, pl.ds(q * 16, 16)] = buf[r, pl.ds(q * 16, 16)] * bc
            return 0

        lax.fori_loop(0, QR // 4, _row4, 0)

    def _rowq(p, _):
        for s in range(4):
            s2 = (s + 2) % 4
            _wait_g(s)
            _scaleq(p, s * QR, s)
            pltpu.async_copy(_slotq(s), acc.at[dqs[s]], sss[s], add=True)
            if s < 2:
                @pl.when(p > 0)
                def _():
                    _wait_s(s2)
                _prep_q(p, s2 * QR, dqs[s2], sqs[s2])
                pltpu.async_copy(h_hbm.at[sqs[s2]], _slotq(s2), gss[s2])
            else:
                _wait_s(s2)

                @pl.when(p + 1 < NCH)
                def _():
                    _prep_q(p + 1, s2 * QR, dqs[s2], sqs[s2])
                    pltpu.async_copy(h_hbm.at[sqs[s2]], _slotq(s2), gss[s2])
        return 0

    lax.fori_loop(0, NCH, _rowq, 0)
    _wait_s(2)
    _wait_s(3)

    plsc.subcore_barrier()
    base_w = jnp.minimum(base, N - ROWS_PT)
    pltpu.sync_copy(acc.at[pl.ds(base_w, ROWS_PT)],
                    out_hbm.at[pl.ds(base_w, ROWS_PT), pl.ds(c * HALF, HALF)])


def kernel(x, edge_index, weight, att):
    a = att.reshape(-1)
    a_pad = (jnp.zeros((OUT_CH, HALF), jnp.float32)
             .at[:, 0].set(a[:OUT_CH])
             .at[:, 1].set(a[OUT_CH:]))
    h2, s_pad = _tc_mm(x, weight, a_pad)
    h_flat = h2.reshape(2 * N, HALF)
    sd = jnp.pad(s_pad[:, 0], (0, N_PAD - N))
    ss = jnp.pad(s_pad[:, 1], (0, N_PAD - N))
    return _sc_gat(h_flat, sd, ss, edge_index[0], edge_index[1])


# speedup vs baseline: 1.0170x; 1.0170x over previous
import functools

import jax
import jax.numpy as jnp
from jax import lax
from jax.experimental import pallas as pl
from jax.experimental.pallas import tpu as pltpu
from jax.experimental.pallas import tpu_sc as plsc

IN_CH = 256
OUT_CH = 256
N = 10000
E = 160000
NEG = 0.2

HALF = 128
NTILES = 16
CH = 128
HC = 64
NCH = 80
EPT = NCH * CH
EPT_REAL = E // NTILES
N_PAD = 10112
ROWS_PT = N_PAD // NTILES
BM = 1000
NEG_BIG = -1e30


def _tc_body(x_ref, w_ref, a_ref, h2_ref, s_ref):
    xb = x_ref[...]
    w = w_ref[...]
    hb = lax.dot_general(xb, w, (((1,), (0,)), ((), ())),
                         preferred_element_type=jnp.float32,
                         precision=lax.Precision.HIGHEST)
    h2_ref[0, ...] = hb[:, :HALF]
    h2_ref[1, ...] = hb[:, HALF:]
    wa = lax.dot_general(w, a_ref[...], (((1,), (0,)), ((), ())),
                         preferred_element_type=jnp.float32,
                         precision=lax.Precision.HIGHEST)
    s_ref[...] = lax.dot_general(xb, wa, (((1,), (0,)), ((), ())),
                                 preferred_element_type=jnp.float32,
                                 precision=lax.Precision.HIGHEST)


_tc_mm = pl.pallas_call(
    _tc_body,
    grid=(N // BM,),
    in_specs=[
        pl.BlockSpec((BM, IN_CH), lambda i: (i, 0)),
        pl.BlockSpec((IN_CH, OUT_CH), lambda i: (0, 0)),
        pl.BlockSpec((OUT_CH, HALF), lambda i: (0, 0)),
    ],
    out_specs=[
        pl.BlockSpec((2, BM, HALF), lambda i: (0, i, 0)),
        pl.BlockSpec((BM, HALF), lambda i: (i, 0)),
    ],
    out_shape=[
        jax.ShapeDtypeStruct((2, N, HALF), jnp.float32),
        jax.ShapeDtypeStruct((N, HALF), jnp.float32),
    ],
)


def _i16(v):
    return jnp.full((16,), v, jnp.int32)


@functools.partial(
    pl.kernel,
    mesh=plsc.VectorSubcoreMesh(core_axis_name="c", subcore_axis_name="s"),
    compiler_params=pltpu.CompilerParams(needs_layout_passes=False),
    out_type=jax.ShapeDtypeStruct((N, 2 * HALF), jnp.float32),
    scratch_types=[
        pltpu.VMEM((EPT,), jnp.int32),
        pltpu.VMEM((EPT,), jnp.int32),
        pltpu.VMEM((EPT,), jnp.float32),
        pltpu.VMEM((CH, HALF), jnp.float32),
        pltpu.VMEM((HC,), jnp.int32),
        pltpu.VMEM((HC,), jnp.int32),
        pltpu.VMEM((HC,), jnp.int32),
        pltpu.VMEM((HC,), jnp.int32),
        pltpu.VMEM((16,), jnp.float32),
        pltpu.VMEM((NTILES * 16,), jnp.float32),
        pltpu.VMEM_SHARED((N_PAD, HALF), jnp.float32),
        pltpu.VMEM_SHARED((NTILES * 16,), jnp.float32),
        pltpu.VMEM_SHARED((NTILES * 16,), jnp.float32),
        pltpu.SemaphoreType.DMA,
        pltpu.SemaphoreType.DMA,
        pltpu.SemaphoreType.DMA,
        pltpu.SemaphoreType.DMA,
        pltpu.SemaphoreType.DMA,
    ],
)
def _sc_gat(h_hbm, sd_hbm, ss_hbm, dst_hbm, src_hbm, out_hbm,
            dst_v, src_v, coef_v, buf, dst0, dst1, src0, src1, pub, redv,
            acc, redm, reds, asem, gsem0, gsem1, ssem0, ssem1):
    c = lax.axis_index("c")
    t = lax.axis_index("s")

    pltpu.sync_copy(dst_hbm.at[pl.ds(t * EPT_REAL, EPT_REAL)],
                    dst_v.at[pl.ds(0, EPT_REAL)])
    pltpu.sync_copy(src_hbm.at[pl.ds(t * EPT_REAL, EPT_REAL)],
                    src_v.at[pl.ds(0, EPT_REAL)])
    for w in range((EPT - EPT_REAL) // 16):
        dst_v[pl.ds(EPT_REAL + w * 16, 16)] = _i16(N_PAD - 1)
        src_v[pl.ds(EPT_REAL + w * 16, 16)] = _i16(0)

    zf = jnp.full((16,), 0.0, jnp.float32)

    def _zero_buf(k, _):
        for q in range(HALF // 16):
            buf[k, pl.ds(q * 16, 16)] = zf
        return 0

    lax.fori_loop(0, CH, _zero_buf, 0)

    base = t * ROWS_PT
    nfull = ROWS_PT // CH
    rem = ROWS_PT % CH

    def _zero_acc(k, _):
        pltpu.sync_copy(buf, acc.at[pl.ds(base + k * CH, CH)])
        return 0

    lax.fori_loop(0, nfull, _zero_acc, 0)
    if rem:
        pltpu.sync_copy(buf.at[pl.ds(0, rem)],
                        acc.at[pl.ds(base + nfull * CH, rem)])

    lanes = lax.iota(jnp.int32, 16)
    gbase = t * EPT

    def _fire_scores(j, _):
        pltpu.async_copy(sd_hbm.at[dst_v.at[pl.ds(j * CH, CH)]],
                         coef_v.at[pl.ds(j * CH, CH)], asem)
        pltpu.async_copy(ss_hbm.at[src_v.at[pl.ds(j * CH, CH)]],
                         buf.at[j], asem)
        return 0

    lax.fori_loop(0, NCH, _fire_scores, 0)

    def _drain_scores(j, _):
        pltpu.make_async_copy(sd_hbm.at[pl.ds(0, CH)],
                              coef_v.at[pl.ds(j * CH, CH)], asem).wait()
        pltpu.make_async_copy(ss_hbm.at[pl.ds(0, CH)], buf.at[j],
                              asem).wait()
        return 0

    lax.fori_loop(0, NCH, _drain_scores, 0)

    def _score_chunk(j, mx):
        for i in range(CH // 16):
            a = (coef_v[pl.ds(j * CH + i * 16, 16)] +
                 buf[j, pl.ds(i * 16, 16)])
            a = jnp.where(a >= 0.0, a, NEG * a)
            gid = _i16(j * CH + i * 16) + lanes
            a = jnp.where(gid < EPT_REAL, a, NEG_BIG)
            coef_v[pl.ds(j * CH + i * 16, 16)] = a
            mx = jnp.maximum(mx, a)
        return mx

    mx = lax.fori_loop(0, NCH, _score_chunk,
                       jnp.full((16,), NEG_BIG, jnp.float32))

    pub[...] = mx
    pltpu.sync_copy(pub, redm.at[pl.ds(t * 16, 16)])
    plsc.subcore_barrier()
    pltpu.sync_copy(redm, redv)
    m16 = redv[pl.ds(0, 16)]
    for i in range(1, NTILES):
        m16 = jnp.maximum(m16, redv[pl.ds(i * 16, 16)])
    mvec = jnp.full((16,), jnp.max(m16))

    def _exp_chunk(j, sm):
        for i in range(CH // 16):
            e = jnp.exp(coef_v[pl.ds(j * CH + i * 16, 16)] - mvec)
            coef_v[pl.ds(j * CH + i * 16, 16)] = e
            sm = sm + e
        return sm

    sm = lax.fori_loop(0, NCH, _exp_chunk, jnp.full((16,), 0.0, jnp.float32))

    pub[...] = sm
    pltpu.sync_copy(pub, reds.at[pl.ds(t * 16, 16)])
    plsc.subcore_barrier()
    pltpu.sync_copy(reds, redv)
    s16 = redv[pl.ds(0, 16)]
    for i in range(1, NTILES):
        s16 = s16 + redv[pl.ds(i * 16, 16)]
    ivec = jnp.full((16,), 1.0, jnp.float32) / jnp.full((16,), jnp.sum(s16))

    offv = _i16(c * N)

    def _fix_chunk(j, _):
        for i in range(CH // 16):
            o = j * CH + i * 16
            coef_v[pl.ds(o, 16)] = coef_v[pl.ds(o, 16)] * ivec
            src_v[pl.ds(o, 16)] = src_v[pl.ds(o, 16)] + offv
        return 0

    lax.fori_loop(0, NCH, _fix_chunk, 0)

    def _prep_idx(p, half, dstX, srcX):
        for v in range(HC // 16):
            o = p * CH + half * HC + v * 16
            dstX[pl.ds(v * 16, 16)] = dst_v[pl.ds(o, 16)]
            srcX[pl.ds(v * 16, 16)] = src_v[pl.ds(o, 16)]

    def _slot(s):
        return buf.at[pl.ds(s * HC, HC)]

    def _wait_gather(s, gsem):
        pltpu.make_async_copy(h_hbm.at[pl.ds(0, HC)], _slot(s), gsem).wait()

    def _wait_scatter(ssem):
        pltpu.make_async_copy(h_hbm.at[pl.ds(0, HC)], _slot(0), ssem).wait()

    def _scale(p, half, s):
        def _row4(k4, _):
            k = k4 * 4
            for d in range(4):
                bc = plsc.load_gather(coef_v,
                                      [_i16(p * CH + half * HC + k + d)])
                r = s * HC + k + d
                for q in range(HALF // 16):
                    buf[r, pl.ds(q * 16, 16)] = buf[r, pl.ds(q * 16, 16)] * bc
            return 0

        lax.fori_loop(0, HC // 4, _row4, 0)

    _prep_idx(0, 0, dst0, src0)
    pltpu.async_copy(h_hbm.at[src0], _slot(0), gsem0)

    def _pair(p, _):
        @pl.when(p > 0)
        def _():
            _wait_scatter(ssem1)
        _prep_idx(p, 1, dst1, src1)
        pltpu.async_copy(h_hbm.at[src1], _slot(1), gsem1)

        _wait_gather(0, gsem0)
        _scale(p, 0, 0)
        pltpu.async_copy(_slot(0), acc.at[dst0], ssem0, add=True)

        _wait_gather(1, gsem1)
        _scale(p, 1, 1)
        pltpu.async_copy(_slot(1), acc.at[dst1], ssem1, add=True)

        _wait_scatter(ssem0)

        @pl.when(p + 1 < NCH)
        def _():
            _prep_idx(p + 1, 0, dst0, src0)
            pltpu.async_copy(h_hbm.at[src0], _slot(0), gsem0)

        return 0

    lax.fori_loop(0, NCH, _pair, 0)
    _wait_scatter(ssem1)

    plsc.subcore_barrier()
    base_w = jnp.minimum(base, N - ROWS_PT)
    pltpu.sync_copy(acc.at[pl.ds(base_w, ROWS_PT)],
                    out_hbm.at[pl.ds(base_w, ROWS_PT), pl.ds(c * HALF, HALF)])


def kernel(x, edge_index, weight, att):
    a = att.reshape(-1)
    a_pad = (jnp.zeros((OUT_CH, HALF), jnp.float32)
             .at[:, 0].set(a[:OUT_CH])
             .at[:, 1].set(a[OUT_CH:]))
    h2, s_pad = _tc_mm(x, weight, a_pad)
    h_flat = h2.reshape(2 * N, HALF)
    sd = jnp.pad(s_pad[:, 0], (0, N_PAD - N))
    ss = jnp.pad(s_pad[:, 1], (0, N_PAD - N))
    return _sc_gat(h_flat, sd, ss, edge_index[0], edge_index[1])
